# Initial kernel scaffold; baseline (speedup 1.0000x reference)
import jax
import jax.numpy as jnp
from jax.experimental import pallas as pl


def kernel(x, edge_index, W1, b1, W2, b2):
    raise NotImplementedError("write your pallas kernel here")


# speedup vs baseline: 10.6458x; 10.6458x over previous
import functools

import jax
import jax.numpy as jnp
from jax import lax
from jax.experimental import pallas as pl
from jax.experimental.pallas import tpu as pltpu
from jax.experimental.pallas import tpu_sc as plsc

N = 10000
E = 320000
F_IN = 128
HIDDEN = 64
N_CLASSES = 40
CPAD = 48

NC = 2
NS = 16
NW = NC * NS
L = 16

N_PAD = 10240
EA = E + N
EP = 331776
CE = 2048
NCHUNK = (EP // NC) // CE
NVEC = CE // L
DEG_W = EP // NW

f32 = jnp.float32
i32 = jnp.int32


@functools.cache
def _mesh():
    return plsc.VectorSubcoreMesh(
        core_axis_name="c", subcore_axis_name="s",
        num_cores=NC, num_subcores=NS)


_SC_PARAMS = pltpu.CompilerParams(needs_layout_passes=False)


def _deg_body(dst_hbm, zeros_hbm, out, didx, dacc, sem):
    c = lax.axis_index("c")
    s = lax.axis_index("s")
    wid = c * NS + s
    pltpu.sync_copy(zeros_hbm, dacc)
    pltpu.sync_copy(dst_hbm.at[pl.ds(wid * DEG_W, DEG_W)], didx)
    ones16 = jnp.full((L,), 1.0, f32)
    row16 = jnp.zeros((L,), i32)

    def step(v, carry):
        d16 = didx[pl.ds(v * L, L)]
        plsc.addupdate_scatter(dacc, [row16, d16], ones16)
        return carry

    lax.fori_loop(0, DEG_W // L, step, 0)
    pltpu.sync_copy(dacc, out.at[wid])


@functools.cache
def _deg_call():
    return pl.kernel(
        _deg_body,
        out_type=jax.ShapeDtypeStruct((NW, 1, N_PAD), f32),
        mesh=_mesh(),
        compiler_params=_SC_PARAMS,
        scratch_types=[
            pltpu.VMEM((DEG_W,), i32),
            pltpu.VMEM((1, N_PAD), f32),
            pltpu.SemaphoreType.DMA,
        ],
    )


def _scat_body(tab_hbm, src_hbm, dst_hbm, dinv_hbm, zeros_hbm, out,
               sidx, didx, tab_v, acc_v, dinv_v, sem, D, WC):
    c = lax.axis_index("c")
    s = lax.axis_index("s")
    pltpu.sync_copy(tab_hbm.at[s], tab_v)
    pltpu.sync_copy(zeros_hbm, acc_v)
    pltpu.sync_copy(dinv_hbm, dinv_v)

    ebase = c * (EP // NC)

    def chunk(k, carry):
        pltpu.sync_copy(src_hbm.at[pl.ds(ebase + k * CE, CE)], sidx)
        pltpu.sync_copy(dst_hbm.at[pl.ds(ebase + k * CE, CE)], didx)

        def step(v, carry2):
            s16 = sidx[pl.ds(v * L, L)]
            d16 = didx[pl.ds(v * L, L)]
            w16 = (plsc.load_gather(dinv_v, [s16])
                   * plsc.load_gather(dinv_v, [d16]))
            for cc in range(WC):
                c16 = jnp.full((L,), cc, i32)
                val = plsc.load_gather(tab_v, [c16, s16])
                plsc.addupdate_scatter(acc_v, [c16, d16], val * w16)
            return carry2

        lax.fori_loop(0, NVEC, step, 0)
        return carry

    lax.fori_loop(0, NCHUNK, chunk, 0)
    pltpu.sync_copy(acc_v, out.at[c, s])


@functools.cache
def _scat_call(D, WC):
    return pl.kernel(
        functools.partial(_scat_body, D=D, WC=WC),
        out_type=jax.ShapeDtypeStruct((NC, NS, WC, N_PAD), f32),
        mesh=_mesh(),
        compiler_params=_SC_PARAMS,
        scratch_types=[
            pltpu.VMEM((CE,), i32),
            pltpu.VMEM((CE,), i32),
            pltpu.VMEM((WC, N_PAD), f32),
            pltpu.VMEM((WC, N_PAD), f32),
            pltpu.VMEM((N_PAD,), f32),
            pltpu.SemaphoreType.DMA,
        ],
    )


_RL = 2048
_GRID = N_PAD // _RL


def _tca_body(deg_ref, x_ref, w1_ref, ht_ref, dinv_ref):
    deg = jnp.sum(deg_ref[...], axis=0)
    dinv_ref[...] = lax.rsqrt(jnp.maximum(deg, 1.0))
    ht = lax.dot_general(
        w1_ref[...], x_ref[...], (((0,), (1,)), ((), ())),
        preferred_element_type=f32, precision=lax.Precision.HIGHEST)
    wc = HIDDEN // NS
    for k in range(NS):
        ht_ref[k] = ht[k * wc:(k + 1) * wc, :]


def _tca(deg_parts, x, W1):
    return pl.pallas_call(
        _tca_body,
        grid=(_GRID,),
        in_specs=[
            pl.BlockSpec((NW, 1, _RL), lambda i: (0, 0, i)),
            pl.BlockSpec((_RL, F_IN), lambda i: (i, 0)),
            pl.BlockSpec((F_IN, HIDDEN), lambda i: (0, 0)),
        ],
        out_specs=[
            pl.BlockSpec((NS, HIDDEN // NS, _RL), lambda i: (0, 0, i)),
            pl.BlockSpec((1, _RL), lambda i: (0, i)),
        ],
        out_shape=[
            jax.ShapeDtypeStruct((NS, HIDDEN // NS, N_PAD), f32),
            jax.ShapeDtypeStruct((1, N_PAD), f32),
        ],
    )(deg_parts, x, W1)


def _tcb_body(s1_ref, b1_ref, w2_ref, emb_ref, h2_ref):
    pre = s1_ref[0] + s1_ref[1] + b1_ref[...]
    emb = jnp.maximum(pre, 0.0)
    emb_ref[...] = emb
    h2 = lax.dot_general(
        w2_ref[...], emb, (((0,), (0,)), ((), ())),
        preferred_element_type=f32, precision=lax.Precision.HIGHEST)
    wc = CPAD // NS
    for k in range(NS):
        h2_ref[k] = h2[k * wc:(k + 1) * wc, :]


def _tcb(s1_parts, b1, W2p):
    return pl.pallas_call(
        _tcb_body,
        grid=(_GRID,),
        in_specs=[
            pl.BlockSpec((NC, HIDDEN, _RL), lambda i: (0, 0, i)),
            pl.BlockSpec((HIDDEN, 1), lambda i: (0, 0)),
            pl.BlockSpec((HIDDEN, CPAD), lambda i: (0, 0)),
        ],
        out_specs=[
            pl.BlockSpec((HIDDEN, _RL), lambda i: (0, i)),
            pl.BlockSpec((NS, CPAD // NS, _RL), lambda i: (0, 0, i)),
        ],
        out_shape=[
            jax.ShapeDtypeStruct((HIDDEN, N_PAD), f32),
            jax.ShapeDtypeStruct((NS, CPAD // NS, N_PAD), f32),
        ],
    )(s1_parts, b1, W2p)


def _tcc_body(s2_ref, b2_ref, out_ref):
    h2 = (s2_ref[0, :N_CLASSES, :] + s2_ref[1, :N_CLASSES, :]
          + b2_ref[...])
    m = jnp.max(h2, axis=0, keepdims=True)
    lse = jnp.log(jnp.sum(jnp.exp(h2 - m), axis=0, keepdims=True)) + m
    out_ref[...] = h2 - lse


def _tcc(s2_parts, b2):
    return pl.pallas_call(
        _tcc_body,
        grid=(_GRID,),
        in_specs=[
            pl.BlockSpec((NC, CPAD, _RL), lambda i: (0, 0, i)),
            pl.BlockSpec((N_CLASSES, 1), lambda i: (0, 0)),
        ],
        out_specs=pl.BlockSpec((N_CLASSES, _RL), lambda i: (0, i)),
        out_shape=jax.ShapeDtypeStruct((N_CLASSES, N_PAD), f32),
    )(s2_parts, b2)


def kernel(x, edge_index, W1, b1, W2, b2):
    src = edge_index[0].astype(i32)
    dst = edge_index[1].astype(i32)
    loop = jnp.arange(N, dtype=i32)
    npad = EP - EA
    pad_src = jnp.arange(npad, dtype=i32) % 8
    pad_dst = N + jnp.arange(npad, dtype=i32) % (N_PAD - N)
    srcp = jnp.concatenate([src, loop, pad_src])
    dstp = jnp.concatenate([dst, loop, pad_dst])

    zeros1 = jnp.zeros((1, N_PAD), f32)
    zeros_h = jnp.zeros((HIDDEN // NS, N_PAD), f32)
    zeros_c = jnp.zeros((CPAD // NS, N_PAD), f32)
    W2p = jnp.pad(W2, ((0, 0), (0, CPAD - N_CLASSES)))

    deg_parts = _deg_call()(dstp, zeros1)
    h1t, dinv = _tca(deg_parts, x, W1)
    dinv_flat = dinv.reshape(N_PAD)
    s1_parts = _scat_call(HIDDEN, HIDDEN // NS)(
        h1t, srcp, dstp, dinv_flat, zeros_h).reshape(NC, HIDDEN, N_PAD)
    embt, h2t = _tcb(s1_parts, b1.reshape(HIDDEN, 1), W2p)
    s2_parts = _scat_call(CPAD, CPAD // NS)(
        h2t, srcp, dstp, dinv_flat, zeros_c).reshape(NC, CPAD, N_PAD)
    logpt = _tcc(s2_parts, b2.reshape(N_CLASSES, 1))
    return (logpt[:, :N].T, embt[:, :N].T)


# speedup vs baseline: 11.3998x; 1.0708x over previous
import functools

import jax
import jax.numpy as jnp
from jax import lax
from jax.experimental import pallas as pl
from jax.experimental.pallas import tpu as pltpu
from jax.experimental.pallas import tpu_sc as plsc

N = 10000
E = 320000
F_IN = 128
HIDDEN = 64
N_CLASSES = 40
CPAD = 48

NC = 2
NS = 16
NW = NC * NS
L = 16

N_PAD = 10240
EA = E + N
EP = 331776
CE = 2048
NCHUNK = (EP // NC) // CE
NVEC = CE // L
DEG_W = EP // NW

f32 = jnp.float32
i32 = jnp.int32


@functools.cache
def _mesh():
    return plsc.VectorSubcoreMesh(
        core_axis_name="c", subcore_axis_name="s",
        num_cores=NC, num_subcores=NS)


_SC_PARAMS = pltpu.CompilerParams(needs_layout_passes=False)


def _deg_body(dst_hbm, zeros_hbm, out, didx, dacc, sem):
    c = lax.axis_index("c")
    s = lax.axis_index("s")
    wid = c * NS + s
    pltpu.sync_copy(zeros_hbm, dacc)
    pltpu.sync_copy(dst_hbm.at[pl.ds(wid * DEG_W, DEG_W)], didx)
    ones16 = jnp.full((L,), 1.0, f32)
    row16 = jnp.zeros((L,), i32)

    def step(v, carry):
        d16 = didx[pl.ds(v * L, L)]
        plsc.addupdate_scatter(dacc, [row16, d16], ones16)
        return carry

    lax.fori_loop(0, DEG_W // L, step, 0)
    pltpu.sync_copy(dacc, out.at[wid])


@functools.cache
def _deg_call():
    return pl.kernel(
        _deg_body,
        out_type=jax.ShapeDtypeStruct((NW, 1, N_PAD), f32),
        mesh=_mesh(),
        compiler_params=_SC_PARAMS,
        scratch_types=[
            pltpu.VMEM((DEG_W,), i32),
            pltpu.VMEM((1, N_PAD), f32),
            pltpu.SemaphoreType.DMA,
        ],
    )


def _scat_body(tab_hbm, src_hbm, dst_hbm, dinv_hbm, zeros_hbm, out,
               sidx, didx, tab_v, acc_v, dinv_v, sem, D, WC):
    c = lax.axis_index("c")
    s = lax.axis_index("s")
    pltpu.sync_copy(tab_hbm.at[s], tab_v)
    pltpu.sync_copy(zeros_hbm, acc_v)
    pltpu.sync_copy(dinv_hbm, dinv_v)

    def scale_tab(j, carry):
        sl = pl.ds(j * L, L)
        dv = dinv_v[sl]
        for cc in range(WC):
            tab_v[cc, sl] = tab_v[cc, sl] * dv
        return carry

    lax.fori_loop(0, N_PAD // L, scale_tab, 0)

    ebase = c * (EP // NC)
    UN = 8

    def chunk(k, carry):
        pltpu.sync_copy(src_hbm.at[pl.ds(ebase + k * CE, CE)], sidx)
        pltpu.sync_copy(dst_hbm.at[pl.ds(ebase + k * CE, CE)], didx)

        def step(v, carry2):
            for t in range(UN):
                sl = pl.ds(v * (L * UN) + t * L, L)
                s16 = sidx[sl]
                d16 = didx[sl]
                for cc in range(WC):
                    c16 = jnp.full((L,), cc, i32)
                    val = plsc.load_gather(tab_v, [c16, s16])
                    plsc.addupdate_scatter(acc_v, [c16, d16], val)
            return carry2

        lax.fori_loop(0, NVEC // UN, step, 0)
        return carry

    lax.fori_loop(0, NCHUNK, chunk, 0)

    def scale_acc(j, carry):
        sl = pl.ds(j * L, L)
        dv = dinv_v[sl]
        for cc in range(WC):
            acc_v[cc, sl] = acc_v[cc, sl] * dv
        return carry

    lax.fori_loop(0, N_PAD // L, scale_acc, 0)
    pltpu.sync_copy(acc_v, out.at[c, s])


@functools.cache
def _scat_call(D, WC):
    return pl.kernel(
        functools.partial(_scat_body, D=D, WC=WC),
        out_type=jax.ShapeDtypeStruct((NC, NS, WC, N_PAD), f32),
        mesh=_mesh(),
        compiler_params=_SC_PARAMS,
        scratch_types=[
            pltpu.VMEM((CE,), i32),
            pltpu.VMEM((CE,), i32),
            pltpu.VMEM((WC, N_PAD), f32),
            pltpu.VMEM((WC, N_PAD), f32),
            pltpu.VMEM((N_PAD,), f32),
            pltpu.SemaphoreType.DMA,
        ],
    )


_RL = 2048
_GRID = N_PAD // _RL


def _tca_body(deg_ref, x_ref, w1_ref, ht_ref, dinv_ref):
    deg = jnp.sum(deg_ref[...], axis=0)
    dinv_ref[...] = lax.rsqrt(jnp.maximum(deg, 1.0))
    ht = lax.dot_general(
        w1_ref[...], x_ref[...], (((0,), (1,)), ((), ())),
        preferred_element_type=f32, precision=lax.Precision.HIGHEST)
    wc = HIDDEN // NS
    for k in range(NS):
        ht_ref[k] = ht[k * wc:(k + 1) * wc, :]


def _tca(deg_parts, x, W1):
    return pl.pallas_call(
        _tca_body,
        grid=(_GRID,),
        in_specs=[
            pl.BlockSpec((NW, 1, _RL), lambda i: (0, 0, i)),
            pl.BlockSpec((_RL, F_IN), lambda i: (i, 0)),
            pl.BlockSpec((F_IN, HIDDEN), lambda i: (0, 0)),
        ],
        out_specs=[
            pl.BlockSpec((NS, HIDDEN // NS, _RL), lambda i: (0, 0, i)),
            pl.BlockSpec((1, _RL), lambda i: (0, i)),
        ],
        out_shape=[
            jax.ShapeDtypeStruct((NS, HIDDEN // NS, N_PAD), f32),
            jax.ShapeDtypeStruct((1, N_PAD), f32),
        ],
    )(deg_parts, x, W1)


def _tcb_body(s1_ref, b1_ref, w2_ref, emb_ref, h2_ref):
    pre = s1_ref[0] + s1_ref[1] + b1_ref[...]
    emb = jnp.maximum(pre, 0.0)
    emb_ref[...] = emb
    h2 = lax.dot_general(
        w2_ref[...], emb, (((0,), (0,)), ((), ())),
        preferred_element_type=f32, precision=lax.Precision.HIGHEST)
    wc = CPAD // NS
    for k in range(NS):
        h2_ref[k] = h2[k * wc:(k + 1) * wc, :]


def _tcb(s1_parts, b1, W2p):
    return pl.pallas_call(
        _tcb_body,
        grid=(_GRID,),
        in_specs=[
            pl.BlockSpec((NC, HIDDEN, _RL), lambda i: (0, 0, i)),
            pl.BlockSpec((HIDDEN, 1), lambda i: (0, 0)),
            pl.BlockSpec((HIDDEN, CPAD), lambda i: (0, 0)),
        ],
        out_specs=[
            pl.BlockSpec((HIDDEN, _RL), lambda i: (0, i)),
            pl.BlockSpec((NS, CPAD // NS, _RL), lambda i: (0, 0, i)),
        ],
        out_shape=[
            jax.ShapeDtypeStruct((HIDDEN, N_PAD), f32),
            jax.ShapeDtypeStruct((NS, CPAD // NS, N_PAD), f32),
        ],
    )(s1_parts, b1, W2p)


def _tcc_body(s2_ref, b2_ref, out_ref):
    h2 = (s2_ref[0, :N_CLASSES, :] + s2_ref[1, :N_CLASSES, :]
          + b2_ref[...])
    m = jnp.max(h2, axis=0, keepdims=True)
    lse = jnp.log(jnp.sum(jnp.exp(h2 - m), axis=0, keepdims=True)) + m
    out_ref[...] = h2 - lse


def _tcc(s2_parts, b2):
    return pl.pallas_call(
        _tcc_body,
        grid=(_GRID,),
        in_specs=[
            pl.BlockSpec((NC, CPAD, _RL), lambda i: (0, 0, i)),
            pl.BlockSpec((N_CLASSES, 1), lambda i: (0, 0)),
        ],
        out_specs=pl.BlockSpec((N_CLASSES, _RL), lambda i: (0, i)),
        out_shape=jax.ShapeDtypeStruct((N_CLASSES, N_PAD), f32),
    )(s2_parts, b2)


def kernel(x, edge_index, W1, b1, W2, b2):
    src = edge_index[0].astype(i32)
    dst = edge_index[1].astype(i32)
    loop = jnp.arange(N, dtype=i32)
    npad = EP - EA
    pad_src = jnp.arange(npad, dtype=i32) % 8
    pad_dst = N + jnp.arange(npad, dtype=i32) % (N_PAD - N)
    srcp = jnp.concatenate([src, loop, pad_src])
    dstp = jnp.concatenate([dst, loop, pad_dst])

    zeros1 = jnp.zeros((1, N_PAD), f32)
    zeros_h = jnp.zeros((HIDDEN // NS, N_PAD), f32)
    zeros_c = jnp.zeros((CPAD // NS, N_PAD), f32)
    W2p = jnp.pad(W2, ((0, 0), (0, CPAD - N_CLASSES)))

    deg_parts = _deg_call()(dstp, zeros1)
    h1t, dinv = _tca(deg_parts, x, W1)
    dinv_flat = dinv.reshape(N_PAD)
    s1_parts = _scat_call(HIDDEN, HIDDEN // NS)(
        h1t, srcp, dstp, dinv_flat, zeros_h).reshape(NC, HIDDEN, N_PAD)
    embt, h2t = _tcb(s1_parts, b1.reshape(HIDDEN, 1), W2p)
    s2_parts = _scat_call(CPAD, CPAD // NS)(
        h2t, srcp, dstp, dinv_flat, zeros_c).reshape(NC, CPAD, N_PAD)
    logpt = _tcc(s2_parts, b2.reshape(N_CLASSES, 1))
    return (logpt[:, :N].T, embt[:, :N].T)


# speedup vs baseline: 18.8804x; 1.6562x over previous
import functools

import jax
import jax.numpy as jnp
from jax import lax
from jax.experimental import pallas as pl
from jax.experimental.pallas import tpu as pltpu
from jax.experimental.pallas import tpu_sc as plsc

N = 10000
E = 320000
F_IN = 128
HIDDEN = 64
N_CLASSES = 40
CPAD = 48

NC = 2
NS = 16
NW = NC * NS
L = 16

N_PAD = 10240
EA = E + N
EP = 331776
CE = 2048
NCHUNK = (EP // NC) // CE
NVEC = CE // L
DEG_W = EP // NW

f32 = jnp.float32
i32 = jnp.int32


@functools.cache
def _mesh():
    return plsc.VectorSubcoreMesh(
        core_axis_name="c", subcore_axis_name="s",
        num_cores=NC, num_subcores=NS)


_SC_PARAMS = pltpu.CompilerParams(needs_layout_passes=False)


def _deg_body(dst_hbm, zeros_hbm, out, didx, dacc, sem):
    c = lax.axis_index("c")
    s = lax.axis_index("s")
    wid = c * NS + s
    pltpu.sync_copy(zeros_hbm, dacc)
    pltpu.sync_copy(dst_hbm.at[pl.ds(wid * DEG_W, DEG_W)], didx)
    ones16 = jnp.full((L,), 1.0, f32)
    row16 = jnp.zeros((L,), i32)

    def step(v, carry):
        d16 = didx[pl.ds(v * L, L)]
        plsc.addupdate_scatter(dacc, [row16, d16], ones16)
        return carry

    lax.fori_loop(0, DEG_W // L, step, 0)
    pltpu.sync_copy(dacc, out.at[wid])


@functools.cache
def _deg_call():
    return pl.kernel(
        _deg_body,
        out_type=jax.ShapeDtypeStruct((NW, 1, N_PAD), f32),
        mesh=_mesh(),
        compiler_params=_SC_PARAMS,
        scratch_types=[
            pltpu.VMEM((DEG_W,), i32),
            pltpu.VMEM((1, N_PAD), f32),
            pltpu.SemaphoreType.DMA,
        ],
    )


def _scat_body(tab_hbm, src_hbm, dst_hbm, dinv_hbm, zeros_hbm, out,
               sidx, didx, tab_v, acc_v, dinv_v, sem, D, WC):
    c = lax.axis_index("c")
    s = lax.axis_index("s")
    pltpu.sync_copy(tab_hbm.at[s], tab_v)
    pltpu.sync_copy(zeros_hbm, acc_v)
    pltpu.sync_copy(dinv_hbm, dinv_v)

    def scale_tab(j, carry):
        sl = pl.ds(j * L, L)
        dv = dinv_v[sl]
        for cc in range(WC):
            tab_v[cc, sl] = tab_v[cc, sl] * dv
        return carry

    lax.fori_loop(0, N_PAD // L, scale_tab, 0)

    ebase = c * (EP // NC)

    def chunk(k, carry):
        pltpu.sync_copy(src_hbm.at[pl.ds(ebase + k * CE, CE)], sidx)
        pltpu.sync_copy(dst_hbm.at[pl.ds(ebase + k * CE, CE)], didx)

        @plsc.parallel_loop(0, NVEC, unroll=8)
        def _(v):
            sl = pl.ds(v * L, L)
            s16 = sidx[sl]
            d16 = didx[sl]
            vals = [plsc.load_gather(tab_v, [jnp.full((L,), cc, i32), s16])
                    for cc in range(WC)]
            for cc in range(WC):
                plsc.addupdate_scatter(
                    acc_v, [jnp.full((L,), cc, i32), d16], vals[cc])

        return carry

    lax.fori_loop(0, NCHUNK, chunk, 0)

    def scale_acc(j, carry):
        sl = pl.ds(j * L, L)
        dv = dinv_v[sl]
        for cc in range(WC):
            acc_v[cc, sl] = acc_v[cc, sl] * dv
        return carry

    lax.fori_loop(0, N_PAD // L, scale_acc, 0)
    pltpu.sync_copy(acc_v, out.at[c, s])


@functools.cache
def _scat_call(D, WC):
    return pl.kernel(
        functools.partial(_scat_body, D=D, WC=WC),
        out_type=jax.ShapeDtypeStruct((NC, NS, WC, N_PAD), f32),
        mesh=_mesh(),
        compiler_params=_SC_PARAMS,
        scratch_types=[
            pltpu.VMEM((CE,), i32),
            pltpu.VMEM((CE,), i32),
            pltpu.VMEM((WC, N_PAD), f32),
            pltpu.VMEM((WC, N_PAD), f32),
            pltpu.VMEM((N_PAD,), f32),
            pltpu.SemaphoreType.DMA,
        ],
    )


_RL = 2048
_GRID = N_PAD // _RL


def _tca_body(deg_ref, x_ref, w1_ref, ht_ref, dinv_ref):
    deg = jnp.sum(deg_ref[...], axis=0)
    dinv_ref[...] = lax.rsqrt(jnp.maximum(deg, 1.0))
    ht = lax.dot_general(
        w1_ref[...], x_ref[...], (((0,), (1,)), ((), ())),
        preferred_element_type=f32, precision=lax.Precision.HIGHEST)
    wc = HIDDEN // NS
    for k in range(NS):
        ht_ref[k] = ht[k * wc:(k + 1) * wc, :]


def _tca(deg_parts, x, W1):
    return pl.pallas_call(
        _tca_body,
        grid=(_GRID,),
        in_specs=[
            pl.BlockSpec((NW, 1, _RL), lambda i: (0, 0, i)),
            pl.BlockSpec((_RL, F_IN), lambda i: (i, 0)),
            pl.BlockSpec((F_IN, HIDDEN), lambda i: (0, 0)),
        ],
        out_specs=[
            pl.BlockSpec((NS, HIDDEN // NS, _RL), lambda i: (0, 0, i)),
            pl.BlockSpec((1, _RL), lambda i: (0, i)),
        ],
        out_shape=[
            jax.ShapeDtypeStruct((NS, HIDDEN // NS, N_PAD), f32),
            jax.ShapeDtypeStruct((1, N_PAD), f32),
        ],
    )(deg_parts, x, W1)


def _tcb_body(s1_ref, b1_ref, w2_ref, emb_ref, h2_ref):
    pre = s1_ref[0] + s1_ref[1] + b1_ref[...]
    emb = jnp.maximum(pre, 0.0)
    emb_ref[...] = emb
    h2 = lax.dot_general(
        w2_ref[...], emb, (((0,), (0,)), ((), ())),
        preferred_element_type=f32, precision=lax.Precision.HIGHEST)
    wc = CPAD // NS
    for k in range(NS):
        h2_ref[k] = h2[k * wc:(k + 1) * wc, :]


def _tcb(s1_parts, b1, W2p):
    return pl.pallas_call(
        _tcb_body,
        grid=(_GRID,),
        in_specs=[
            pl.BlockSpec((NC, HIDDEN, _RL), lambda i: (0, 0, i)),
            pl.BlockSpec((HIDDEN, 1), lambda i: (0, 0)),
            pl.BlockSpec((HIDDEN, CPAD), lambda i: (0, 0)),
        ],
        out_specs=[
            pl.BlockSpec((HIDDEN, _RL), lambda i: (0, i)),
            pl.BlockSpec((NS, CPAD // NS, _RL), lambda i: (0, 0, i)),
        ],
        out_shape=[
            jax.ShapeDtypeStruct((HIDDEN, N_PAD), f32),
            jax.ShapeDtypeStruct((NS, CPAD // NS, N_PAD), f32),
        ],
    )(s1_parts, b1, W2p)


def _tcc_body(s2_ref, b2_ref, out_ref):
    h2 = (s2_ref[0, :N_CLASSES, :] + s2_ref[1, :N_CLASSES, :]
          + b2_ref[...])
    m = jnp.max(h2, axis=0, keepdims=True)
    lse = jnp.log(jnp.sum(jnp.exp(h2 - m), axis=0, keepdims=True)) + m
    out_ref[...] = h2 - lse


def _tcc(s2_parts, b2):
    return pl.pallas_call(
        _tcc_body,
        grid=(_GRID,),
        in_specs=[
            pl.BlockSpec((NC, CPAD, _RL), lambda i: (0, 0, i)),
            pl.BlockSpec((N_CLASSES, 1), lambda i: (0, 0)),
        ],
        out_specs=pl.BlockSpec((N_CLASSES, _RL), lambda i: (0, i)),
        out_shape=jax.ShapeDtypeStruct((N_CLASSES, N_PAD), f32),
    )(s2_parts, b2)


def kernel(x, edge_index, W1, b1, W2, b2):
    src = edge_index[0].astype(i32)
    dst = edge_index[1].astype(i32)
    loop = jnp.arange(N, dtype=i32)
    npad = EP - EA
    pad_src = jnp.arange(npad, dtype=i32) % 8
    pad_dst = N + jnp.arange(npad, dtype=i32) % (N_PAD - N)
    srcp = jnp.concatenate([src, loop, pad_src])
    dstp = jnp.concatenate([dst, loop, pad_dst])

    zeros1 = jnp.zeros((1, N_PAD), f32)
    zeros_h = jnp.zeros((HIDDEN // NS, N_PAD), f32)
    zeros_c = jnp.zeros((CPAD // NS, N_PAD), f32)
    W2p = jnp.pad(W2, ((0, 0), (0, CPAD - N_CLASSES)))

    deg_parts = _deg_call()(dstp, zeros1)
    h1t, dinv = _tca(deg_parts, x, W1)
    dinv_flat = dinv.reshape(N_PAD)
    s1_parts = _scat_call(HIDDEN, HIDDEN // NS)(
        h1t, srcp, dstp, dinv_flat, zeros_h).reshape(NC, HIDDEN, N_PAD)
    embt, h2t = _tcb(s1_parts, b1.reshape(HIDDEN, 1), W2p)
    s2_parts = _scat_call(CPAD, CPAD // NS)(
        h2t, srcp, dstp, dinv_flat, zeros_c).reshape(NC, CPAD, N_PAD)
    logpt = _tcc(s2_parts, b2.reshape(N_CLASSES, 1))
    return (logpt[:, :N].T, embt[:, :N].T)


# speedup vs baseline: 26.2186x; 1.3887x over previous
import functools

import jax
import jax.numpy as jnp
from jax import lax
from jax.experimental import pallas as pl
from jax.experimental.pallas import tpu as pltpu
from jax.experimental.pallas import tpu_sc as plsc

N = 10000
E = 320000
F_IN = 128
HIDDEN = 64
N_CLASSES = 40
CPAD = 48

NC = 2
NS = 16
NW = NC * NS
L = 16

N_PAD = 10240
EA = E + N
EP = 331776
CE = 10368
NCHUNK = (EP // NC) // CE
NVEC = CE // L
DEG_W = EP // NW

f32 = jnp.float32
i32 = jnp.int32


@functools.cache
def _mesh():
    return plsc.VectorSubcoreMesh(
        core_axis_name="c", subcore_axis_name="s",
        num_cores=NC, num_subcores=NS)


_SC_PARAMS = pltpu.CompilerParams(needs_layout_passes=False)


def _deg_body(dst_hbm, zeros_hbm, out, didx, dacc, sem):
    c = lax.axis_index("c")
    s = lax.axis_index("s")
    wid = c * NS + s
    pltpu.sync_copy(zeros_hbm, dacc)
    pltpu.sync_copy(dst_hbm.at[pl.ds(wid * DEG_W, DEG_W)], didx)
    ones16 = jnp.full((L,), 1.0, f32)
    row16 = jnp.zeros((L,), i32)

    def step(v, carry):
        d16 = didx[pl.ds(v * L, L)]
        plsc.addupdate_scatter(dacc, [row16, d16], ones16)
        return carry

    lax.fori_loop(0, DEG_W // L, step, 0)
    pltpu.sync_copy(dacc, out.at[wid])


@functools.cache
def _deg_call():
    return pl.kernel(
        _deg_body,
        out_type=jax.ShapeDtypeStruct((NW, 1, N_PAD), f32),
        mesh=_mesh(),
        compiler_params=_SC_PARAMS,
        scratch_types=[
            pltpu.VMEM((DEG_W,), i32),
            pltpu.VMEM((1, N_PAD), f32),
            pltpu.SemaphoreType.DMA,
        ],
    )


def _scat_body(tab_hbm, src_hbm, dst_hbm, dinv_hbm, zeros_hbm, out,
               sidx, didx, tab_v, acc_v, dinv_v, sem, D, WC):
    c = lax.axis_index("c")
    s = lax.axis_index("s")
    pltpu.sync_copy(tab_hbm.at[s], tab_v)
    pltpu.sync_copy(zeros_hbm, acc_v)
    pltpu.sync_copy(dinv_hbm, dinv_v)

    def scale_tab(j, carry):
        sl = pl.ds(j * L, L)
        dv = dinv_v[sl]
        for cc in range(WC):
            tab_v[cc, sl] = tab_v[cc, sl] * dv
        return carry

    lax.fori_loop(0, N_PAD // L, scale_tab, 0)

    ebase = c * (EP // NC)

    def chunk(k, carry):
        pltpu.sync_copy(src_hbm.at[pl.ds(ebase + k * CE, CE)], sidx)
        pltpu.sync_copy(dst_hbm.at[pl.ds(ebase + k * CE, CE)], didx)

        @plsc.parallel_loop(0, NVEC, unroll=8)
        def _(v):
            sl = pl.ds(v * L, L)
            s16 = sidx[sl]
            d16 = didx[sl]
            vals = [plsc.load_gather(tab_v, [jnp.full((L,), cc, i32), s16])
                    for cc in range(WC)]
            for cc in range(WC):
                plsc.addupdate_scatter(
                    acc_v, [jnp.full((L,), cc, i32), d16], vals[cc])

        return carry

    lax.fori_loop(0, NCHUNK, chunk, 0)

    def scale_acc(j, carry):
        sl = pl.ds(j * L, L)
        dv = dinv_v[sl]
        for cc in range(WC):
            acc_v[cc, sl] = acc_v[cc, sl] * dv
        return carry

    lax.fori_loop(0, N_PAD // L, scale_acc, 0)
    pltpu.sync_copy(acc_v, out.at[c, s])


@functools.cache
def _scat_call(D, WC):
    return pl.kernel(
        functools.partial(_scat_body, D=D, WC=WC),
        out_type=jax.ShapeDtypeStruct((NC, NS, WC, N_PAD), f32),
        mesh=_mesh(),
        compiler_params=_SC_PARAMS,
        scratch_types=[
            pltpu.VMEM((CE,), i32),
            pltpu.VMEM((CE,), i32),
            pltpu.VMEM((WC, N_PAD), f32),
            pltpu.VMEM((WC, N_PAD), f32),
            pltpu.VMEM((N_PAD,), f32),
            pltpu.SemaphoreType.DMA,
        ],
    )


_RL = 2048
_GRID = N_PAD // _RL


def _tca_body(deg_ref, x_ref, w1_ref, ht_ref, dinv_ref):
    deg = jnp.sum(deg_ref[...], axis=0)
    dinv_ref[...] = lax.rsqrt(jnp.maximum(deg, 1.0))
    ht = lax.dot_general(
        w1_ref[...], x_ref[...], (((0,), (1,)), ((), ())),
        preferred_element_type=f32, precision=lax.Precision.HIGHEST)
    wc = HIDDEN // NS
    for k in range(NS):
        ht_ref[k] = ht[k * wc:(k + 1) * wc, :]


def _tca(deg_parts, x, W1):
    return pl.pallas_call(
        _tca_body,
        grid=(_GRID,),
        in_specs=[
            pl.BlockSpec((NW, 1, _RL), lambda i: (0, 0, i)),
            pl.BlockSpec((_RL, F_IN), lambda i: (i, 0)),
            pl.BlockSpec((F_IN, HIDDEN), lambda i: (0, 0)),
        ],
        out_specs=[
            pl.BlockSpec((NS, HIDDEN // NS, _RL), lambda i: (0, 0, i)),
            pl.BlockSpec((1, _RL), lambda i: (0, i)),
        ],
        out_shape=[
            jax.ShapeDtypeStruct((NS, HIDDEN // NS, N_PAD), f32),
            jax.ShapeDtypeStruct((1, N_PAD), f32),
        ],
    )(deg_parts, x, W1)


def _tcb_body(s1_ref, b1_ref, w2_ref, emb_ref, h2_ref):
    pre = s1_ref[0] + s1_ref[1] + b1_ref[...]
    emb = jnp.maximum(pre, 0.0)
    emb_ref[...] = emb
    h2 = lax.dot_general(
        w2_ref[...], emb, (((0,), (0,)), ((), ())),
        preferred_element_type=f32, precision=lax.Precision.HIGHEST)
    wc = CPAD // NS
    for k in range(NS):
        h2_ref[k] = h2[k * wc:(k + 1) * wc, :]


def _tcb(s1_parts, b1, W2p):
    return pl.pallas_call(
        _tcb_body,
        grid=(_GRID,),
        in_specs=[
            pl.BlockSpec((NC, HIDDEN, _RL), lambda i: (0, 0, i)),
            pl.BlockSpec((HIDDEN, 1), lambda i: (0, 0)),
            pl.BlockSpec((HIDDEN, CPAD), lambda i: (0, 0)),
        ],
        out_specs=[
            pl.BlockSpec((HIDDEN, _RL), lambda i: (0, i)),
            pl.BlockSpec((NS, CPAD // NS, _RL), lambda i: (0, 0, i)),
        ],
        out_shape=[
            jax.ShapeDtypeStruct((HIDDEN, N_PAD), f32),
            jax.ShapeDtypeStruct((NS, CPAD // NS, N_PAD), f32),
        ],
    )(s1_parts, b1, W2p)


def _tcc_body(s2_ref, b2_ref, out_ref):
    h2 = (s2_ref[0, :N_CLASSES, :] + s2_ref[1, :N_CLASSES, :]
          + b2_ref[...])
    m = jnp.max(h2, axis=0, keepdims=True)
    lse = jnp.log(jnp.sum(jnp.exp(h2 - m), axis=0, keepdims=True)) + m
    out_ref[...] = h2 - lse


def _tcc(s2_parts, b2):
    return pl.pallas_call(
        _tcc_body,
        grid=(_GRID,),
        in_specs=[
            pl.BlockSpec((NC, CPAD, _RL), lambda i: (0, 0, i)),
            pl.BlockSpec((N_CLASSES, 1), lambda i: (0, 0)),
        ],
        out_specs=pl.BlockSpec((N_CLASSES, _RL), lambda i: (0, i)),
        out_shape=jax.ShapeDtypeStruct((N_CLASSES, N_PAD), f32),
    )(s2_parts, b2)


def kernel(x, edge_index, W1, b1, W2, b2):
    src = edge_index[0].astype(i32)
    dst = edge_index[1].astype(i32)
    loop = jnp.arange(N, dtype=i32)
    npad = EP - EA
    pad_src = jnp.arange(npad, dtype=i32) % 8
    pad_dst = N + jnp.arange(npad, dtype=i32) % (N_PAD - N)
    srcp = jnp.concatenate([src, loop, pad_src])
    dstp = jnp.concatenate([dst, loop, pad_dst])

    zeros1 = jnp.zeros((1, N_PAD), f32)
    zeros_h = jnp.zeros((HIDDEN // NS, N_PAD), f32)
    zeros_c = jnp.zeros((CPAD // NS, N_PAD), f32)
    W2p = jnp.pad(W2, ((0, 0), (0, CPAD - N_CLASSES)))

    deg_parts = _deg_call()(dstp, zeros1)
    h1t, dinv = _tca(deg_parts, x, W1)
    dinv_flat = dinv.reshape(N_PAD)
    s1_parts = _scat_call(HIDDEN, HIDDEN // NS)(
        h1t, srcp, dstp, dinv_flat, zeros_h).reshape(NC, HIDDEN, N_PAD)
    embt, h2t = _tcb(s1_parts, b1.reshape(HIDDEN, 1), W2p)
    s2_parts = _scat_call(CPAD, CPAD // NS)(
        h2t, srcp, dstp, dinv_flat, zeros_c).reshape(NC, CPAD, N_PAD)
    logpt = _tcc(s2_parts, b2.reshape(N_CLASSES, 1))
    return (logpt[:, :N].T, embt[:, :N].T)


# speedup vs baseline: 31.3831x; 1.1970x over previous
import functools

import jax
import jax.numpy as jnp
from jax import lax
from jax.experimental import pallas as pl
from jax.experimental.pallas import tpu as pltpu
from jax.experimental.pallas import tpu_sc as plsc

N = 10000
E = 320000
F_IN = 128
HIDDEN = 64
N_CLASSES = 40
CPAD = 48

NC = 2
NS = 16
NW = NC * NS
L = 16

N_PAD = 10240
EA = E + N
EP = 331776
CE = 5184
NCHUNK = (EP // NC) // CE
NVEC = CE // L
DEG_W = EP // NW

f32 = jnp.float32
i32 = jnp.int32


@functools.cache
def _mesh():
    return plsc.VectorSubcoreMesh(
        core_axis_name="c", subcore_axis_name="s",
        num_cores=NC, num_subcores=NS)


_SC_PARAMS = pltpu.CompilerParams(needs_layout_passes=False)


def _deg_body(dst_hbm, zeros_hbm, out, didx, dacc, sem):
    c = lax.axis_index("c")
    s = lax.axis_index("s")
    wid = c * NS + s
    pltpu.sync_copy(zeros_hbm, dacc)
    pltpu.sync_copy(dst_hbm.at[pl.ds(wid * DEG_W, DEG_W)], didx)
    ones16 = jnp.full((L,), 1.0, f32)
    row16 = jnp.zeros((L,), i32)

    def step(v, carry):
        d16 = didx[pl.ds(v * L, L)]
        plsc.addupdate_scatter(dacc, [row16, d16], ones16)
        return carry

    lax.fori_loop(0, DEG_W // L, step, 0)
    pltpu.sync_copy(dacc, out.at[wid])


@functools.cache
def _deg_call():
    return pl.kernel(
        _deg_body,
        out_type=jax.ShapeDtypeStruct((NW, 1, N_PAD), f32),
        mesh=_mesh(),
        compiler_params=_SC_PARAMS,
        scratch_types=[
            pltpu.VMEM((DEG_W,), i32),
            pltpu.VMEM((1, N_PAD), f32),
            pltpu.SemaphoreType.DMA,
        ],
    )


def _scat_body(tab_hbm, src_hbm, dst_hbm, dinv_hbm, zeros_hbm, out,
               sidx0, sidx1, didx0, didx1, tab_v, acc_v, dinv_v,
               ssem0, ssem1, dsem0, dsem1, D, WC):
    c = lax.axis_index("c")
    s = lax.axis_index("s")
    pltpu.sync_copy(tab_hbm.at[s], tab_v)
    pltpu.sync_copy(zeros_hbm, acc_v)
    pltpu.sync_copy(dinv_hbm, dinv_v)

    def scale_tab(j, carry):
        sl = pl.ds(j * L, L)
        dv = dinv_v[sl]
        for cc in range(WC):
            tab_v[cc, sl] = tab_v[cc, sl] * dv
        return carry

    lax.fori_loop(0, N_PAD // L, scale_tab, 0)

    ebase = c * (EP // NC)
    sbufs = (sidx0, sidx1)
    dbufs = (didx0, didx1)
    ssems = (ssem0, ssem1)
    dsems = (dsem0, dsem1)
    for b in range(2):
        pltpu.async_copy(src_hbm.at[pl.ds(ebase + b * CE, CE)],
                         sbufs[b], ssems[b])
        pltpu.async_copy(dst_hbm.at[pl.ds(ebase + b * CE, CE)],
                         dbufs[b], dsems[b])

    def chunk(k2, carry):
        for b in range(2):
            kk = k2 * 2 + b
            base = ebase + kk * CE
            pltpu.make_async_copy(
                src_hbm.at[pl.ds(base, CE)], sbufs[b], ssems[b]).wait()
            pltpu.make_async_copy(
                dst_hbm.at[pl.ds(base, CE)], dbufs[b], dsems[b]).wait()

            sidx = sbufs[b]
            didx = dbufs[b]

            @plsc.parallel_loop(0, NVEC, unroll=8)
            def _(v):
                sl = pl.ds(v * L, L)
                s16 = sidx[sl]
                d16 = didx[sl]
                vals = [plsc.load_gather(tab_v,
                                         [jnp.full((L,), cc, i32), s16])
                        for cc in range(WC)]
                for cc in range(WC):
                    plsc.addupdate_scatter(
                        acc_v, [jnp.full((L,), cc, i32), d16], vals[cc])

            @pl.when(kk < NCHUNK - 2)
            def _():
                nbase = ebase + (kk + 2) * CE
                pltpu.async_copy(src_hbm.at[pl.ds(nbase, CE)],
                                 sbufs[b], ssems[b])
                pltpu.async_copy(dst_hbm.at[pl.ds(nbase, CE)],
                                 dbufs[b], dsems[b])

        return carry

    lax.fori_loop(0, NCHUNK // 2, chunk, 0)

    def scale_acc(j, carry):
        sl = pl.ds(j * L, L)
        dv = dinv_v[sl]
        for cc in range(WC):
            acc_v[cc, sl] = acc_v[cc, sl] * dv
        return carry

    lax.fori_loop(0, N_PAD // L, scale_acc, 0)
    pltpu.sync_copy(acc_v, out.at[c, s])


@functools.cache
def _scat_call(D, WC):
    return pl.kernel(
        functools.partial(_scat_body, D=D, WC=WC),
        out_type=jax.ShapeDtypeStruct((NC, NS, WC, N_PAD), f32),
        mesh=_mesh(),
        compiler_params=_SC_PARAMS,
        scratch_types=[
            pltpu.VMEM((CE,), i32),
            pltpu.VMEM((CE,), i32),
            pltpu.VMEM((CE,), i32),
            pltpu.VMEM((CE,), i32),
            pltpu.VMEM((WC, N_PAD), f32),
            pltpu.VMEM((WC, N_PAD), f32),
            pltpu.VMEM((N_PAD,), f32),
            pltpu.SemaphoreType.DMA,
            pltpu.SemaphoreType.DMA,
            pltpu.SemaphoreType.DMA,
            pltpu.SemaphoreType.DMA,
        ],
    )


_RL = 2048
_GRID = N_PAD // _RL


def _tca_body(deg_ref, x_ref, w1_ref, ht_ref, dinv_ref):
    deg = jnp.sum(deg_ref[...], axis=0)
    dinv_ref[...] = lax.rsqrt(jnp.maximum(deg, 1.0))
    ht = lax.dot_general(
        w1_ref[...], x_ref[...], (((0,), (1,)), ((), ())),
        preferred_element_type=f32, precision=lax.Precision.HIGHEST)
    wc = HIDDEN // NS
    for k in range(NS):
        ht_ref[k] = ht[k * wc:(k + 1) * wc, :]


def _tca(deg_parts, x, W1):
    return pl.pallas_call(
        _tca_body,
        grid=(_GRID,),
        in_specs=[
            pl.BlockSpec((NW, 1, _RL), lambda i: (0, 0, i)),
            pl.BlockSpec((_RL, F_IN), lambda i: (i, 0)),
            pl.BlockSpec((F_IN, HIDDEN), lambda i: (0, 0)),
        ],
        out_specs=[
            pl.BlockSpec((NS, HIDDEN // NS, _RL), lambda i: (0, 0, i)),
            pl.BlockSpec((1, _RL), lambda i: (0, i)),
        ],
        out_shape=[
            jax.ShapeDtypeStruct((NS, HIDDEN // NS, N_PAD), f32),
            jax.ShapeDtypeStruct((1, N_PAD), f32),
        ],
    )(deg_parts, x, W1)


def _tcb_body(s1_ref, b1_ref, w2_ref, emb_ref, h2_ref):
    pre = s1_ref[0] + s1_ref[1] + b1_ref[...]
    emb = jnp.maximum(pre, 0.0)
    emb_ref[...] = emb
    h2 = lax.dot_general(
        w2_ref[...], emb, (((0,), (0,)), ((), ())),
        preferred_element_type=f32, precision=lax.Precision.HIGHEST)
    wc = CPAD // NS
    for k in range(NS):
        h2_ref[k] = h2[k * wc:(k + 1) * wc, :]


def _tcb(s1_parts, b1, W2p):
    return pl.pallas_call(
        _tcb_body,
        grid=(_GRID,),
        in_specs=[
            pl.BlockSpec((NC, HIDDEN, _RL), lambda i: (0, 0, i)),
            pl.BlockSpec((HIDDEN, 1), lambda i: (0, 0)),
            pl.BlockSpec((HIDDEN, CPAD), lambda i: (0, 0)),
        ],
        out_specs=[
            pl.BlockSpec((HIDDEN, _RL), lambda i: (0, i)),
            pl.BlockSpec((NS, CPAD // NS, _RL), lambda i: (0, 0, i)),
        ],
        out_shape=[
            jax.ShapeDtypeStruct((HIDDEN, N_PAD), f32),
            jax.ShapeDtypeStruct((NS, CPAD // NS, N_PAD), f32),
        ],
    )(s1_parts, b1, W2p)


def _tcc_body(s2_ref, b2_ref, out_ref):
    h2 = (s2_ref[0, :N_CLASSES, :] + s2_ref[1, :N_CLASSES, :]
          + b2_ref[...])
    m = jnp.max(h2, axis=0, keepdims=True)
    lse = jnp.log(jnp.sum(jnp.exp(h2 - m), axis=0, keepdims=True)) + m
    out_ref[...] = h2 - lse


def _tcc(s2_parts, b2):
    return pl.pallas_call(
        _tcc_body,
        grid=(_GRID,),
        in_specs=[
            pl.BlockSpec((NC, CPAD, _RL), lambda i: (0, 0, i)),
            pl.BlockSpec((N_CLASSES, 1), lambda i: (0, 0)),
        ],
        out_specs=pl.BlockSpec((N_CLASSES, _RL), lambda i: (0, i)),
        out_shape=jax.ShapeDtypeStruct((N_CLASSES, N_PAD), f32),
    )(s2_parts, b2)


def kernel(x, edge_index, W1, b1, W2, b2):
    src = edge_index[0].astype(i32)
    dst = edge_index[1].astype(i32)
    loop = jnp.arange(N, dtype=i32)
    npad = EP - EA
    pad_src = jnp.arange(npad, dtype=i32) % 8
    pad_dst = N + jnp.arange(npad, dtype=i32) % (N_PAD - N)
    srcp = jnp.concatenate([src, loop, pad_src])
    dstp = jnp.concatenate([dst, loop, pad_dst])

    zeros1 = jnp.zeros((1, N_PAD), f32)
    zeros_h = jnp.zeros((HIDDEN // NS, N_PAD), f32)
    zeros_c = jnp.zeros((CPAD // NS, N_PAD), f32)
    W2p = jnp.pad(W2, ((0, 0), (0, CPAD - N_CLASSES)))

    deg_parts = _deg_call()(dstp, zeros1)
    h1t, dinv = _tca(deg_parts, x, W1)
    dinv_flat = dinv.reshape(N_PAD)
    s1_parts = _scat_call(HIDDEN, HIDDEN // NS)(
        h1t, srcp, dstp, dinv_flat, zeros_h).reshape(NC, HIDDEN, N_PAD)
    embt, h2t = _tcb(s1_parts, b1.reshape(HIDDEN, 1), W2p)
    s2_parts = _scat_call(CPAD, CPAD // NS)(
        h2t, srcp, dstp, dinv_flat, zeros_c).reshape(NC, CPAD, N_PAD)
    logpt = _tcc(s2_parts, b2.reshape(N_CLASSES, 1))
    return (logpt[:, :N].T, embt[:, :N].T)


# speedup vs baseline: 31.5754x; 1.0061x over previous
import functools

import jax
import jax.numpy as jnp
from jax import lax
from jax.experimental import pallas as pl
from jax.experimental.pallas import tpu as pltpu
from jax.experimental.pallas import tpu_sc as plsc

N = 10000
E = 320000
F_IN = 128
HIDDEN = 64
N_CLASSES = 40
CPAD = 48

NC = 2
NS = 16
NW = NC * NS
L = 16

N_PAD = 10240
EA = E + N
EP = 331776
CE = 5184
NCHUNK = (EP // NC) // CE
NVEC = CE // L
DEG_W = EP // NW

f32 = jnp.float32
i32 = jnp.int32


@functools.cache
def _mesh():
    return plsc.VectorSubcoreMesh(
        core_axis_name="c", subcore_axis_name="s",
        num_cores=NC, num_subcores=NS)


_SC_PARAMS = pltpu.CompilerParams(needs_layout_passes=False)


def _deg_body(dst_hbm, zeros_hbm, out, didx, dacc, sem):
    c = lax.axis_index("c")
    s = lax.axis_index("s")
    wid = c * NS + s
    pltpu.sync_copy(zeros_hbm, dacc)
    pltpu.sync_copy(dst_hbm.at[pl.ds(wid * DEG_W, DEG_W)], didx)
    ones16 = jnp.full((L,), 1.0, f32)
    row16 = jnp.zeros((L,), i32)

    def step(v, carry):
        d16 = didx[pl.ds(v * L, L)]
        plsc.addupdate_scatter(dacc, [row16, d16], ones16)
        return carry

    lax.fori_loop(0, DEG_W // L, step, 0)
    pltpu.sync_copy(dacc, out.at[wid])


@functools.cache
def _deg_call():
    return pl.kernel(
        _deg_body,
        out_type=jax.ShapeDtypeStruct((NW, 1, N_PAD), f32),
        mesh=_mesh(),
        compiler_params=_SC_PARAMS,
        scratch_types=[
            pltpu.VMEM((DEG_W,), i32),
            pltpu.VMEM((1, N_PAD), f32),
            pltpu.SemaphoreType.DMA,
        ],
    )


def _scat_body(tab_hbm, src_hbm, dst_hbm, dinv_hbm, zeros_hbm, out,
               sidx0, sidx1, didx0, didx1, tab_v, acc_v, dinv_v,
               ssem0, ssem1, dsem0, dsem1, D, WC):
    c = lax.axis_index("c")
    s = lax.axis_index("s")
    pltpu.sync_copy(tab_hbm.at[s], tab_v)
    pltpu.sync_copy(zeros_hbm, acc_v)
    pltpu.sync_copy(dinv_hbm, dinv_v)

    def scale_tab(j, carry):
        sl = pl.ds(j * L, L)
        dv = dinv_v[sl]
        for cc in range(WC):
            tab_v[cc, sl] = tab_v[cc, sl] * dv
        return carry

    lax.fori_loop(0, N_PAD // L, scale_tab, 0)

    ebase = c * (EP // NC)
    sbufs = (sidx0, sidx1)
    dbufs = (didx0, didx1)
    ssems = (ssem0, ssem1)
    dsems = (dsem0, dsem1)
    for b in range(2):
        pltpu.async_copy(src_hbm.at[pl.ds(ebase + b * CE, CE)],
                         sbufs[b], ssems[b])
        pltpu.async_copy(dst_hbm.at[pl.ds(ebase + b * CE, CE)],
                         dbufs[b], dsems[b])

    def chunk(k2, carry):
        for b in range(2):
            kk = k2 * 2 + b
            base = ebase + kk * CE
            pltpu.make_async_copy(
                src_hbm.at[pl.ds(base, CE)], sbufs[b], ssems[b]).wait()
            pltpu.make_async_copy(
                dst_hbm.at[pl.ds(base, CE)], dbufs[b], dsems[b]).wait()

            sidx = sbufs[b]
            didx = dbufs[b]

            @plsc.parallel_loop(0, NVEC, unroll=16)
            def _(v):
                sl = pl.ds(v * L, L)
                s16 = sidx[sl]
                d16 = didx[sl]
                vals = [plsc.load_gather(tab_v,
                                         [jnp.full((L,), cc, i32), s16])
                        for cc in range(WC)]
                for cc in range(WC):
                    plsc.addupdate_scatter(
                        acc_v, [jnp.full((L,), cc, i32), d16], vals[cc])

            @pl.when(kk < NCHUNK - 2)
            def _():
                nbase = ebase + (kk + 2) * CE
                pltpu.async_copy(src_hbm.at[pl.ds(nbase, CE)],
                                 sbufs[b], ssems[b])
                pltpu.async_copy(dst_hbm.at[pl.ds(nbase, CE)],
                                 dbufs[b], dsems[b])

        return carry

    lax.fori_loop(0, NCHUNK // 2, chunk, 0)

    def scale_acc(j, carry):
        sl = pl.ds(j * L, L)
        dv = dinv_v[sl]
        for cc in range(WC):
            acc_v[cc, sl] = acc_v[cc, sl] * dv
        return carry

    lax.fori_loop(0, N_PAD // L, scale_acc, 0)
    pltpu.sync_copy(acc_v, out.at[c, s])


@functools.cache
def _scat_call(D, WC):
    return pl.kernel(
        functools.partial(_scat_body, D=D, WC=WC),
        out_type=jax.ShapeDtypeStruct((NC, NS, WC, N_PAD), f32),
        mesh=_mesh(),
        compiler_params=_SC_PARAMS,
        scratch_types=[
            pltpu.VMEM((CE,), i32),
            pltpu.VMEM((CE,), i32),
            pltpu.VMEM((CE,), i32),
            pltpu.VMEM((CE,), i32),
            pltpu.VMEM((WC, N_PAD), f32),
            pltpu.VMEM((WC, N_PAD), f32),
            pltpu.VMEM((N_PAD,), f32),
            pltpu.SemaphoreType.DMA,
            pltpu.SemaphoreType.DMA,
            pltpu.SemaphoreType.DMA,
            pltpu.SemaphoreType.DMA,
        ],
    )


_RL = 2048
_GRID = N_PAD // _RL


def _tca_body(deg_ref, x_ref, w1_ref, ht_ref, dinv_ref):
    deg = jnp.sum(deg_ref[...], axis=0)
    dinv_ref[...] = lax.rsqrt(jnp.maximum(deg, 1.0))
    ht = lax.dot_general(
        w1_ref[...], x_ref[...], (((0,), (1,)), ((), ())),
        preferred_element_type=f32, precision=lax.Precision.HIGHEST)
    wc = HIDDEN // NS
    for k in range(NS):
        ht_ref[k] = ht[k * wc:(k + 1) * wc, :]


def _tca(deg_parts, x, W1):
    return pl.pallas_call(
        _tca_body,
        grid=(_GRID,),
        in_specs=[
            pl.BlockSpec((NW, 1, _RL), lambda i: (0, 0, i)),
            pl.BlockSpec((_RL, F_IN), lambda i: (i, 0)),
            pl.BlockSpec((F_IN, HIDDEN), lambda i: (0, 0)),
        ],
        out_specs=[
            pl.BlockSpec((NS, HIDDEN // NS, _RL), lambda i: (0, 0, i)),
            pl.BlockSpec((1, _RL), lambda i: (0, i)),
        ],
        out_shape=[
            jax.ShapeDtypeStruct((NS, HIDDEN // NS, N_PAD), f32),
            jax.ShapeDtypeStruct((1, N_PAD), f32),
        ],
    )(deg_parts, x, W1)


def _tcb_body(s1_ref, b1_ref, w2_ref, emb_ref, h2_ref):
    pre = s1_ref[0] + s1_ref[1] + b1_ref[...]
    emb = jnp.maximum(pre, 0.0)
    emb_ref[...] = emb
    h2 = lax.dot_general(
        w2_ref[...], emb, (((0,), (0,)), ((), ())),
        preferred_element_type=f32, precision=lax.Precision.HIGHEST)
    wc = CPAD // NS
    for k in range(NS):
        h2_ref[k] = h2[k * wc:(k + 1) * wc, :]


def _tcb(s1_parts, b1, W2p):
    return pl.pallas_call(
        _tcb_body,
        grid=(_GRID,),
        in_specs=[
            pl.BlockSpec((NC, HIDDEN, _RL), lambda i: (0, 0, i)),
            pl.BlockSpec((HIDDEN, 1), lambda i: (0, 0)),
            pl.BlockSpec((HIDDEN, CPAD), lambda i: (0, 0)),
        ],
        out_specs=[
            pl.BlockSpec((HIDDEN, _RL), lambda i: (0, i)),
            pl.BlockSpec((NS, CPAD // NS, _RL), lambda i: (0, 0, i)),
        ],
        out_shape=[
            jax.ShapeDtypeStruct((HIDDEN, N_PAD), f32),
            jax.ShapeDtypeStruct((NS, CPAD // NS, N_PAD), f32),
        ],
    )(s1_parts, b1, W2p)


def _tcc_body(s2_ref, b2_ref, out_ref):
    h2 = (s2_ref[0, :N_CLASSES, :] + s2_ref[1, :N_CLASSES, :]
          + b2_ref[...])
    m = jnp.max(h2, axis=0, keepdims=True)
    lse = jnp.log(jnp.sum(jnp.exp(h2 - m), axis=0, keepdims=True)) + m
    out_ref[...] = h2 - lse


def _tcc(s2_parts, b2):
    return pl.pallas_call(
        _tcc_body,
        grid=(_GRID,),
        in_specs=[
            pl.BlockSpec((NC, CPAD, _RL), lambda i: (0, 0, i)),
            pl.BlockSpec((N_CLASSES, 1), lambda i: (0, 0)),
        ],
        out_specs=pl.BlockSpec((N_CLASSES, _RL), lambda i: (0, i)),
        out_shape=jax.ShapeDtypeStruct((N_CLASSES, N_PAD), f32),
    )(s2_parts, b2)


def kernel(x, edge_index, W1, b1, W2, b2):
    src = edge_index[0].astype(i32)
    dst = edge_index[1].astype(i32)
    loop = jnp.arange(N, dtype=i32)
    npad = EP - EA
    pad_src = jnp.arange(npad, dtype=i32) % 8
    pad_dst = N + jnp.arange(npad, dtype=i32) % (N_PAD - N)
    srcp = jnp.concatenate([src, loop, pad_src])
    dstp = jnp.concatenate([dst, loop, pad_dst])

    zeros1 = jnp.zeros((1, N_PAD), f32)
    zeros_h = jnp.zeros((HIDDEN // NS, N_PAD), f32)
    zeros_c = jnp.zeros((CPAD // NS, N_PAD), f32)
    W2p = jnp.pad(W2, ((0, 0), (0, CPAD - N_CLASSES)))

    deg_parts = _deg_call()(dstp, zeros1)
    h1t, dinv = _tca(deg_parts, x, W1)
    dinv_flat = dinv.reshape(N_PAD)
    s1_parts = _scat_call(HIDDEN, HIDDEN // NS)(
        h1t, srcp, dstp, dinv_flat, zeros_h).reshape(NC, HIDDEN, N_PAD)
    embt, h2t = _tcb(s1_parts, b1.reshape(HIDDEN, 1), W2p)
    s2_parts = _scat_call(CPAD, CPAD // NS)(
        h2t, srcp, dstp, dinv_flat, zeros_c).reshape(NC, CPAD, N_PAD)
    logpt = _tcc(s2_parts, b2.reshape(N_CLASSES, 1))
    return (logpt[:, :N].T, embt[:, :N].T)
